# Initial kernel scaffold; baseline (speedup 1.0000x reference)
import jax
import jax.numpy as jnp
from jax.experimental import pallas as pl


def kernel(x, perm):
    raise NotImplementedError("write your pallas kernel here")


# speedup vs baseline: 2.3892x; 2.3892x over previous
import functools

import jax
import jax.numpy as jnp
from jax import lax
from jax.experimental import pallas as pl
from jax.experimental.pallas import tpu as pltpu
from jax.experimental.pallas import tpu_sc as plsc

_B, _S, _D = 4, 4096, 2048
_NC, _NS = 2, 16
_NW = _NC * _NS
_ROWS = _B * _S
_RPW = _ROWS // _NW
_CHUNK = 16
_NCHUNK = _RPW // _CHUNK
_LANES = 16


def _gather_body(x_hbm, perm_hbm, out_hbm, idx_v, buf_v, sem):
    cid = lax.axis_index("c")
    sid = lax.axis_index("s")
    wid = sid * _NC + cid
    base = wid * _RPW
    b = base // _S
    i0 = base - b * _S
    off = b * _S

    pltpu.sync_copy(perm_hbm.at[pl.ds(i0, _RPW)], idx_v)
    off_vec = jnp.full((_LANES,), off, dtype=jnp.int32)
    for j in range(_RPW // _LANES):
        sl = pl.ds(j * _LANES, _LANES)
        idx_v[sl] = idx_v[sl] + off_vec

    def chunk(g, carry):
        idx_slice = idx_v.at[pl.ds(g * _CHUNK, _CHUNK)]
        pltpu.async_copy(x_hbm.at[idx_slice], buf_v, sem).wait()
        pltpu.sync_copy(buf_v, out_hbm.at[pl.ds(base + g * _CHUNK, _CHUNK)])
        return carry

    lax.fori_loop(0, _NCHUNK, chunk, 0)


@jax.jit
def kernel(x, perm):
    x2 = x.reshape(_ROWS, _D)
    p32 = perm.astype(jnp.int32)
    mesh = plsc.VectorSubcoreMesh(core_axis_name="c", subcore_axis_name="s")
    run = pl.kernel(
        _gather_body,
        mesh=mesh,
        out_type=jax.ShapeDtypeStruct((_ROWS, _D), jnp.float32),
        scratch_types=[
            pltpu.VMEM((_RPW,), jnp.int32),
            pltpu.VMEM((_CHUNK, _D), jnp.float32),
            pltpu.SemaphoreType.DMA,
        ],
    )
    out = run(x2, p32)
    return out.reshape(_B, _S, _D)


# speedup vs baseline: 2.7949x; 1.1698x over previous
import functools

import jax
import jax.numpy as jnp
from jax import lax
from jax.experimental import pallas as pl
from jax.experimental.pallas import tpu as pltpu
from jax.experimental.pallas import tpu_sc as plsc

_B, _S, _D = 4, 4096, 2048
_NC, _NS = 2, 16
_NW = _NC * _NS
_ROWS = _B * _S
_RPW = _ROWS // _NW
_CHUNK = 8
_NBUF = 4
_NCHUNK = _RPW // _CHUNK
_NGROUP = _NCHUNK // _NBUF
_LANES = 16


def _gather_body(x_hbm, perm_hbm, out_hbm, idx_v, buf_v, *sems):
    sem_g = sems[:_NBUF]
    sem_s = sems[_NBUF:]
    cid = lax.axis_index("c")
    sid = lax.axis_index("s")
    wid = sid * _NC + cid
    base = wid * _RPW
    b = base // _S
    i0 = base - b * _S
    off = b * _S

    pltpu.sync_copy(perm_hbm.at[pl.ds(i0, _RPW)], idx_v)
    off_vec = jnp.full((_LANES,), off, dtype=jnp.int32)
    for j in range(_RPW // _LANES):
        sl = pl.ds(j * _LANES, _LANES)
        idx_v[sl] = idx_v[sl] + off_vec

    def g_copy(g, slot):
        idx_slice = idx_v.at[pl.ds(g * _CHUNK, _CHUNK)]
        return pltpu.make_async_copy(
            x_hbm.at[idx_slice], buf_v.at[slot], sem_g[slot])

    def s_copy(g, slot):
        return pltpu.make_async_copy(
            buf_v.at[slot], out_hbm.at[pl.ds(base + g * _CHUNK, _CHUNK)],
            sem_s[slot])

    for s in range(_NBUF):
        g_copy(s, s).start()

    def group(i, carry):
        g0 = i * _NBUF
        for s in range(_NBUF):
            g_copy(g0 + s, s).wait()
            s_copy(g0 + s, s).start()
        for s in range(_NBUF):
            s_copy(g0 + s, s).wait()
            g_copy(g0 + _NBUF + s, s).start()
        return carry

    lax.fori_loop(0, _NGROUP - 1, group, 0)

    gl = (_NGROUP - 1) * _NBUF
    for s in range(_NBUF):
        g_copy(gl + s, s).wait()
        s_copy(gl + s, s).start()
    for s in range(_NBUF):
        s_copy(gl + s, s).wait()


@jax.jit
def kernel(x, perm):
    x2 = x.reshape(_ROWS, _D)
    p32 = perm.astype(jnp.int32)
    mesh = plsc.VectorSubcoreMesh(core_axis_name="c", subcore_axis_name="s")
    run = pl.kernel(
        _gather_body,
        mesh=mesh,
        out_type=jax.ShapeDtypeStruct((_ROWS, _D), jnp.float32),
        scratch_types=[
            pltpu.VMEM((_RPW,), jnp.int32),
            pltpu.VMEM((_NBUF, _CHUNK, _D), jnp.float32),
        ] + [pltpu.SemaphoreType.DMA] * (2 * _NBUF),
    )
    out = run(x2, p32)
    return out.reshape(_B, _S, _D)


# speedup vs baseline: 2.8555x; 1.0217x over previous
import functools

import jax
import jax.numpy as jnp
from jax import lax
from jax.experimental import pallas as pl
from jax.experimental.pallas import tpu as pltpu
from jax.experimental.pallas import tpu_sc as plsc

_B, _S, _D = 4, 4096, 2048
_NC, _NS = 2, 16
_NW = _NC * _NS
_ROWS = _B * _S
_RPW = _ROWS // _NW
_CHUNK = 8
_NBUF = 4
_NCHUNK = _RPW // _CHUNK
_NGROUP = _NCHUNK // _NBUF
_LANES = 16


def _gather_body(x_hbm, perm_hbm, out_hbm, idx_v, buf_v, *sems):
    sem_g = sems[:_NBUF]
    sem_s = sems[_NBUF:]
    cid = lax.axis_index("c")
    sid = lax.axis_index("s")
    wid = sid * _NC + cid
    base = wid * _RPW
    b = base // _S
    i0 = base - b * _S
    off = b * _S

    pltpu.sync_copy(perm_hbm.at[pl.ds(i0, _RPW)], idx_v)
    off_vec = jnp.full((_LANES,), off, dtype=jnp.int32)
    for j in range(_RPW // _LANES):
        sl = pl.ds(j * _LANES, _LANES)
        idx_v[sl] = idx_v[sl] + off_vec

    def g_copy(g, slot):
        idx_slice = idx_v.at[pl.ds(g * _CHUNK, _CHUNK)]
        return pltpu.make_async_copy(
            x_hbm.at[idx_slice], buf_v.at[slot], sem_g[slot])

    def s_copy(g, slot):
        return pltpu.make_async_copy(
            buf_v.at[slot], out_hbm.at[pl.ds(base + g * _CHUNK, _CHUNK)],
            sem_s[slot])

    g_copy(0, 0).start()
    g_copy(1, 1).start()
    g_copy(2, 2).start()
    g_copy(0, 0).wait()
    s_copy(0, 0).start()
    g_copy(3, 3).start()
    g_copy(1, 1).wait()
    s_copy(1, 1).start()

    def steady(t, carry):
        for b in range(_NBUF):
            g = 2 + t * _NBUF + b
            slot_c = (2 + b) % _NBUF
            slot_p = b
            s_copy(g - 2, slot_p).wait()
            g_copy(g + 2, slot_p).start()
            g_copy(g, slot_c).wait()
            s_copy(g, slot_c).start()
        return carry

    lax.fori_loop(0, (_NCHUNK - _NBUF) // _NBUF, steady, 0)

    n = _NCHUNK
    g_copy(n - 2, (n - 2) % _NBUF).wait()
    s_copy(n - 2, (n - 2) % _NBUF).start()
    g_copy(n - 1, (n - 1) % _NBUF).wait()
    s_copy(n - 1, (n - 1) % _NBUF).start()
    for g in range(n - _NBUF, n):
        s_copy(g, g % _NBUF).wait()


@jax.jit
def kernel(x, perm):
    x2 = x.reshape(_ROWS, _D)
    p32 = perm.astype(jnp.int32)
    mesh = plsc.VectorSubcoreMesh(core_axis_name="c", subcore_axis_name="s")
    run = pl.kernel(
        _gather_body,
        mesh=mesh,
        out_type=jax.ShapeDtypeStruct((_ROWS, _D), jnp.float32),
        scratch_types=[
            pltpu.VMEM((_RPW,), jnp.int32),
            pltpu.VMEM((_NBUF, _CHUNK, _D), jnp.float32),
        ] + [pltpu.SemaphoreType.DMA] * (2 * _NBUF),
    )
    out = run(x2, p32)
    return out.reshape(_B, _S, _D)
